# Initial kernel scaffold; baseline (speedup 1.0000x reference)
import jax
import jax.numpy as jnp
from jax.experimental import pallas as pl


def kernel(x, edge_index, W1, b1, W2, b2, temp):
    raise NotImplementedError("write your pallas kernel here")


# speedup vs baseline: 10.7528x; 10.7528x over previous
import functools

import jax
import jax.numpy as jnp
from jax import lax
from jax.experimental import pallas as pl
from jax.experimental.pallas import tpu as pltpu
from jax.experimental.pallas import tpu_sc as plsc

N = 10000
E = 320000
F_IN = 128
HID = 256
C = 40
K = 10

D = 48
NC, NS = 2, 16
NT = NC * NS
N_PAD = 10240
TRASH = N
ESLAB = 10240
E_PAD = NT * ESLAB
CH = 1024
NCHUNK = ESLAB // CH
ZROWS = N_PAD // NS
ROWS_PT = N_PAD // NT


def _mesh():
    return plsc.VectorSubcoreMesh(
        core_axis_name="c", subcore_axis_name="s",
        num_cores=NC, num_subcores=NS)


def _prep(row3, col3):
    @functools.partial(
        pl.kernel,
        out_type=(jax.ShapeDtypeStruct((NT, ESLAB), jnp.int32),
                  jax.ShapeDtypeStruct((NC, N_PAD), jnp.float32)),
        mesh=_mesh(),
        compiler_params=pltpu.CompilerParams(use_tc_tiling_on_sc=False),
        scratch_types=[
            pltpu.VMEM((ESLAB,), jnp.int32),
            pltpu.VMEM((ESLAB,), jnp.int32),
            pltpu.VMEM((ESLAB,), jnp.float32),
            pltpu.VMEM((ZROWS,), jnp.float32),
            pltpu.VMEM_SHARED((N_PAD,), jnp.float32),
        ],
    )
    def prep_k(row_hbm, col_hbm, colp_hbm, degp_hbm,
               row_v, col_v, ones_v, zb_v, deg_sh):
        cid = lax.axis_index("c")
        sid = lax.axis_index("s")
        t = cid * NS + sid

        def zb_body(i, carry):
            zb_v[pl.ds(i * 16, 16)] = jnp.zeros((16,), jnp.float32)
            return carry
        lax.fori_loop(0, ZROWS // 16, zb_body, 0)
        pltpu.sync_copy(zb_v, deg_sh.at[pl.ds(sid * ZROWS, ZROWS)])
        plsc.subcore_barrier()

        pltpu.sync_copy(row_hbm.at[t], row_v)
        pltpu.sync_copy(col_hbm.at[t], col_v)

        def body(i, carry):
            sl = pl.ds(i * 16, 16)
            rr = row_v[sl]
            cv = col_v[sl]
            col_v[sl] = jnp.where(rr == cv, TRASH, cv)
            ones_v[sl] = jnp.ones((16,), jnp.float32)
            return carry
        lax.fori_loop(0, ESLAB // 16, body, 0)

        pltpu.sync_copy(col_v, colp_hbm.at[t])
        pltpu.sync_copy(ones_v, deg_sh.at[col_v], add=True)
        plsc.subcore_barrier()
        pltpu.sync_copy(deg_sh.at[pl.ds(sid * ZROWS, ZROWS)],
                        degp_hbm.at[cid, pl.ds(sid * ZROWS, ZROWS)])

    return prep_k(row3, col3)


def _mlp_init(xp, degT, W1, b1, W2, b2, t0):
    BR = 1024

    def body(x_ref, degT_ref, W1_ref, b1_ref, W2_ref, b2_ref, t0_ref,
             u0_ref, acc_ref, d2_ref):
        xb = x_ref[...]
        h = jnp.maximum(
            jnp.dot(xb, W1_ref[...], preferred_element_type=jnp.float32)
            + b1_ref[...], 0.0)
        z = jnp.dot(h, W2_ref[...], preferred_element_type=jnp.float32) \
            + b2_ref[...]
        deg = degT_ref[:, 0:1] + degT_ref[:, 1:2] + 1.0
        dinv = lax.rsqrt(deg)
        u48 = jnp.concatenate(
            [z * dinv, jnp.zeros((BR, D - C), jnp.float32)], axis=1)
        u0_ref[...] = u48
        acc_ref[...] = t0_ref[0, 0] * u48
        d2_ref[...] = jnp.broadcast_to(dinv * dinv, (BR, D))

    return pl.pallas_call(
        body,
        grid=(N_PAD // BR,),
        in_specs=[
            pl.BlockSpec((BR, F_IN), lambda i: (i, 0)),
            pl.BlockSpec((BR, 2), lambda i: (i, 0)),
            pl.BlockSpec((F_IN, HID), lambda i: (0, 0)),
            pl.BlockSpec((1, HID), lambda i: (0, 0)),
            pl.BlockSpec((HID, C), lambda i: (0, 0)),
            pl.BlockSpec((1, C), lambda i: (0, 0)),
            pl.BlockSpec((1, 1), lambda i: (0, 0)),
        ],
        out_specs=[
            pl.BlockSpec((BR, D), lambda i: (i, 0)),
            pl.BlockSpec((BR, D), lambda i: (i, 0)),
            pl.BlockSpec((BR, D), lambda i: (i, 0)),
        ],
        out_shape=(
            jax.ShapeDtypeStruct((N_PAD, D), jnp.float32),
            jax.ShapeDtypeStruct((N_PAD, D), jnp.float32),
            jax.ShapeDtypeStruct((N_PAD, D), jnp.float32),
        ),
    )(xp, degT, W1, b1, W2, b2, t0)


def _scatter_round(u, row3, colp3):
    @functools.partial(
        pl.kernel,
        out_type=jax.ShapeDtypeStruct((NC, N_PAD, D), jnp.float32),
        mesh=_mesh(),
        compiler_params=pltpu.CompilerParams(use_tc_tiling_on_sc=False),
        scratch_types=[
            pltpu.VMEM((CH,), jnp.int32),
            pltpu.VMEM((CH,), jnp.int32),
            pltpu.VMEM((CH, D), jnp.float32),
            pltpu.VMEM((ZROWS, D), jnp.float32),
            pltpu.VMEM_SHARED((N_PAD, D), jnp.float32),
            pltpu.SemaphoreType.DMA,
        ],
    )
    def scat_k(u_hbm, row_hbm, colp_hbm, s_hbm,
               ridx_v, cidx_v, buf_v, zb_v, s_sh, sem):
        cid = lax.axis_index("c")
        sid = lax.axis_index("s")
        t = cid * NS + sid

        def zb_body(r, carry):
            for cc in range(D // 16):
                zb_v[r, pl.ds(cc * 16, 16)] = jnp.zeros((16,), jnp.float32)
            return carry
        lax.fori_loop(0, ZROWS, zb_body, 0)
        pltpu.sync_copy(zb_v, s_sh.at[pl.ds(sid * ZROWS, ZROWS)])
        plsc.subcore_barrier()

        def chunk(i, carry):
            pltpu.sync_copy(row_hbm.at[t, pl.ds(i * CH, CH)], ridx_v)
            pltpu.async_copy(u_hbm.at[ridx_v], buf_v, sem).wait()
            pltpu.sync_copy(colp_hbm.at[t, pl.ds(i * CH, CH)], cidx_v)
            pltpu.sync_copy(buf_v, s_sh.at[cidx_v], add=True)
            return carry
        lax.fori_loop(0, NCHUNK, chunk, 0)
        plsc.subcore_barrier()

        pltpu.sync_copy(s_sh.at[pl.ds(sid * ZROWS, ZROWS)],
                        s_hbm.at[cid, pl.ds(sid * ZROWS, ZROWS)])

    return scat_k(u, row3, colp3)


def _update_round(s2, u, acc, d2e, tkb):
    @functools.partial(
        pl.kernel,
        out_type=(jax.ShapeDtypeStruct((N_PAD, D), jnp.float32),
                  jax.ShapeDtypeStruct((N_PAD, D), jnp.float32)),
        mesh=_mesh(),
        compiler_params=pltpu.CompilerParams(use_tc_tiling_on_sc=False),
        scratch_types=[
            pltpu.VMEM((ROWS_PT, D), jnp.float32),
            pltpu.VMEM((ROWS_PT, D), jnp.float32),
            pltpu.VMEM((ROWS_PT, D), jnp.float32),
            pltpu.VMEM((ROWS_PT, D), jnp.float32),
            pltpu.VMEM((ROWS_PT, D), jnp.float32),
            pltpu.VMEM((16,), jnp.float32),
        ],
    )
    def upd_k(s_hbm, u_hbm, acc_hbm, d2_hbm, tk_hbm, uo_hbm, ao_hbm,
              s0_v, s1_v, u_v, a_v, d_v, tk_v):
        cid = lax.axis_index("c")
        sid = lax.axis_index("s")
        r0 = (cid * NS + sid) * ROWS_PT
        pltpu.sync_copy(s_hbm.at[0, pl.ds(r0, ROWS_PT)], s0_v)
        pltpu.sync_copy(s_hbm.at[1, pl.ds(r0, ROWS_PT)], s1_v)
        pltpu.sync_copy(u_hbm.at[pl.ds(r0, ROWS_PT)], u_v)
        pltpu.sync_copy(acc_hbm.at[pl.ds(r0, ROWS_PT)], a_v)
        pltpu.sync_copy(d2_hbm.at[pl.ds(r0, ROWS_PT)], d_v)
        pltpu.sync_copy(tk_hbm, tk_v)
        tk = tk_v[...]

        def body(r, carry):
            for cc in range(D // 16):
                sl = pl.ds(cc * 16, 16)
                un = d_v[r, sl] * (s0_v[r, sl] + s1_v[r, sl] + u_v[r, sl])
                u_v[r, sl] = un
                a_v[r, sl] = a_v[r, sl] + tk * un
            return carry
        lax.fori_loop(0, ROWS_PT, body, 0)

        pltpu.sync_copy(u_v, uo_hbm.at[pl.ds(r0, ROWS_PT)])
        pltpu.sync_copy(a_v, ao_hbm.at[pl.ds(r0, ROWS_PT)])

    return upd_k(s2, u, acc, d2e, tkb)


def _final(acc, degT):
    BR = 2000

    def body(acc_ref, degT_ref, out_ref):
        deg = degT_ref[:, 0:1] + degT_ref[:, 1:2] + 1.0
        out_ref[...] = jnp.sqrt(deg) * acc_ref[:, :C]

    return pl.pallas_call(
        body,
        grid=(N // BR,),
        in_specs=[
            pl.BlockSpec((BR, D), lambda i: (i, 0)),
            pl.BlockSpec((BR, 2), lambda i: (i, 0)),
        ],
        out_specs=pl.BlockSpec((BR, C), lambda i: (i, 0)),
        out_shape=jax.ShapeDtypeStruct((N, C), jnp.float32),
    )(acc, degT)


def kernel(x, edge_index, W1, b1, W2, b2, temp):
    row = edge_index[0]
    col = edge_index[1]
    pad_e = E_PAD - E
    rowp = jnp.concatenate(
        [row, jnp.zeros((pad_e,), jnp.int32)]).reshape(NT, ESLAB)
    colp_in = jnp.concatenate(
        [col, jnp.zeros((pad_e,), jnp.int32)]).reshape(NT, ESLAB)
    colp3, degp = _prep(rowp, colp_in)
    degT = jnp.transpose(degp)
    xp = jnp.concatenate(
        [x, jnp.zeros((N_PAD - N, F_IN), jnp.float32)], axis=0)
    u, acc, d2e = _mlp_init(xp, degT, W1, b1.reshape(1, HID), W2,
                            b2.reshape(1, C), temp[0].reshape(1, 1))
    for k in range(K):
        s2 = _scatter_round(u, rowp, colp3)
        tkb = jnp.broadcast_to(temp[k + 1], (16,))
        u, acc = _update_round(s2, u, acc, d2e, tkb)
    return _final(acc, degT)
